# Initial kernel scaffold; baseline (speedup 1.0000x reference)
import jax
import jax.numpy as jnp
from jax.experimental import pallas as pl


def kernel(x, edge_index, edge_weight, W1, W2, W3):
    raise NotImplementedError("write your pallas kernel here")


# speedup vs baseline: 25.1718x; 25.1718x over previous
import functools

import jax
import jax.numpy as jnp
from jax import lax
from jax.experimental import pallas as pl
from jax.experimental.pallas import tpu as pltpu
from jax.experimental.pallas import tpu_sc as plsc

N = 100000
F, C = 128, 3
H1, H2, OUT = 20, 40, 2
E0 = 3200000
LN = 16
SEG = 128
CHUNK = 1024
SUB = CHUNK // SEG
EP = 32 * CHUNK * 98
ROWS_L12 = EP // 16 // SEG
ROWS_L3 = EP // 32 // SEG
NCH_L12 = ROWS_L12 // SUB
NCH_L3 = ROWS_L3 // SUB
NP = 100096
OUT_PT = NP // 16

_f32 = jnp.float32


def _out_blocks():
    blks, o = [], 0
    while o < OUT_PT:
        n = min(CHUNK, OUT_PT - o)
        blks.append((o, n))
        o += n
    return blks


def _fill_rows(rows, n, val):
    def body(i, _):
        rows[i, :] = jnp.full((LN,), val, _f32)
        return 0
    lax.fori_loop(0, n, body, 0)


def _scale_rows(rows, wv):
    def body(i, _):
        w16 = wv[pl.ds(i * LN, LN)]
        e0 = i * LN
        for k in range(LN):
            rows[e0 + k, :] = rows[e0 + k, :] * w16[k]
        return 0
    lax.fori_loop(0, CHUNK // LN, body, 0)


def _relu_rows(rows, n):
    def body(i, _):
        rows[i, :] = jnp.maximum(rows[i, :], 0.0)
        return 0
    lax.fori_loop(0, n, body, 0, unroll=8)


def _spmm_core(src2, dst2, w1d, table, out, srcv, dstv, wv, rows, acc, sem,
               tid, n_chunks, edge_row_base, relu):
    _fill_rows(rows, CHUNK, 0.0)
    ob = tid * OUT_PT
    for (o, n) in _out_blocks():
        pltpu.sync_copy(rows.at[pl.ds(0, n)], acc.at[pl.ds(ob + o, n)])
    plsc.subcore_barrier()

    def chunk(i, _):
        row0 = edge_row_base + i * SUB
        pltpu.sync_copy(src2.at[pl.ds(row0, SUB)], srcv)
        pltpu.sync_copy(dst2.at[pl.ds(row0, SUB)], dstv)
        pltpu.sync_copy(w1d.at[pl.ds(row0 * SEG, CHUNK)], wv)
        cps = [pltpu.async_copy(table.at[srcv.at[j]],
                                rows.at[pl.ds(j * SEG, SEG)], sem)
               for j in range(SUB)]
        for cp in cps:
            cp.wait()
        _scale_rows(rows, wv)
        for j in range(SUB):
            pltpu.sync_copy(rows.at[pl.ds(j * SEG, SEG)],
                            acc.at[dstv.at[j]], add=True)
        return 0

    lax.fori_loop(0, n_chunks, chunk, 0)
    plsc.subcore_barrier()

    if relu:
        for (o, n) in _out_blocks():
            pltpu.sync_copy(acc.at[pl.ds(ob + o, n)], rows.at[pl.ds(0, n)])
            _relu_rows(rows, n)
            pltpu.sync_copy(rows.at[pl.ds(0, n)], out.at[pl.ds(ob + o, n)])
    else:
        pltpu.sync_copy(acc.at[pl.ds(ob, OUT_PT)], out.at[pl.ds(ob, OUT_PT)])


def _sc_scratch():
    return [
        pltpu.VMEM((SUB, SEG), jnp.int32),
        pltpu.VMEM((SUB, SEG), jnp.int32),
        pltpu.VMEM((CHUNK,), _f32),
        pltpu.VMEM((CHUNK, LN), _f32),
        pltpu.VMEM_SHARED((NP, LN), _f32),
        pltpu.SemaphoreType.DMA,
    ]


def _make_spmm12(relu):
    @functools.partial(
        pl.kernel,
        mesh=plsc.VectorSubcoreMesh(core_axis_name="c", subcore_axis_name="s"),
        out_type=[jax.ShapeDtypeStruct((NP, LN), _f32),
                  jax.ShapeDtypeStruct((NP, LN), _f32)],
        scratch_types=_sc_scratch(),
        compiler_params=pltpu.CompilerParams(use_tc_tiling_on_sc=False),
    )
    def k(src2, dst2, w1d, tlo, thi, out_lo, out_hi,
          srcv, dstv, wv, rows, acc, sem):
        c = lax.axis_index("c")
        s = lax.axis_index("s")
        base = s * ROWS_L12

        @pl.when(c == 0)
        def _():
            _spmm_core(src2, dst2, w1d, tlo, out_lo, srcv, dstv, wv, rows,
                       acc, sem, s, NCH_L12, base, relu)

        @pl.when(c == 1)
        def _():
            _spmm_core(src2, dst2, w1d, thi, out_hi, srcv, dstv, wv, rows,
                       acc, sem, s, NCH_L12, base, relu)

    return k


def _make_spmm3():
    @functools.partial(
        pl.kernel,
        mesh=plsc.VectorSubcoreMesh(core_axis_name="c", subcore_axis_name="s"),
        out_type=[jax.ShapeDtypeStruct((NP, LN), _f32),
                  jax.ShapeDtypeStruct((NP, LN), _f32)],
        scratch_types=_sc_scratch(),
        compiler_params=pltpu.CompilerParams(use_tc_tiling_on_sc=False),
    )
    def k(src2, dst2, w1d, tbl, p0, p1, srcv, dstv, wv, rows, acc, sem):
        c = lax.axis_index("c")
        s = lax.axis_index("s")
        base = (c * 16 + s) * ROWS_L3

        @pl.when(c == 0)
        def _():
            _spmm_core(src2, dst2, w1d, tbl, p0, srcv, dstv, wv, rows,
                       acc, sem, s, NCH_L3, base, False)

        @pl.when(c == 1)
        def _():
            _spmm_core(src2, dst2, w1d, tbl, p1, srcv, dstv, wv, rows,
                       acc, sem, s, NCH_L3, base, False)

    return k


_spmm12_relu = _make_spmm12(True)
_spmm12_plain = _make_spmm12(False)
_spmm3 = _make_spmm3()

_B = 1000


def _mm1_body(x_ref, w_ref, lo_ref, hi_ref):
    r = jnp.dot(x_ref[...], w_ref[...], preferred_element_type=_f32,
                precision=lax.Precision.HIGHEST)
    lo_ref[...] = r[:, :LN]
    hi_ref[...] = r[:, LN:]


_mm1 = pl.pallas_call(
    _mm1_body,
    grid=(N // _B,),
    in_specs=[pl.BlockSpec((_B, F * C), lambda i: (i, 0)),
              pl.BlockSpec((F * C, 2 * LN), lambda i: (0, 0))],
    out_specs=[pl.BlockSpec((_B, LN), lambda i: (i, 0)),
               pl.BlockSpec((_B, LN), lambda i: (i, 0))],
    out_shape=[jax.ShapeDtypeStruct((N, LN), _f32),
               jax.ShapeDtypeStruct((N, LN), _f32)],
)


def _mm2_body(lo_ref, hi_ref, w2_ref, w3_ref, g_ref):
    cat = jnp.concatenate([lo_ref[...], hi_ref[...]], axis=1)
    t = jnp.maximum(jnp.dot(cat, w2_ref[...], preferred_element_type=_f32,
                            precision=lax.Precision.HIGHEST), 0.0)
    g_ref[...] = jnp.dot(t, w3_ref[...], preferred_element_type=_f32,
                         precision=lax.Precision.HIGHEST)


_mm2 = pl.pallas_call(
    _mm2_body,
    grid=(N // _B,),
    in_specs=[pl.BlockSpec((_B, LN), lambda i: (i, 0)),
              pl.BlockSpec((_B, LN), lambda i: (i, 0)),
              pl.BlockSpec((2 * LN, H2), lambda i: (0, 0)),
              pl.BlockSpec((H2, LN), lambda i: (0, 0))],
    out_specs=pl.BlockSpec((_B, LN), lambda i: (i, 0)),
    out_shape=jax.ShapeDtypeStruct((N, LN), _f32),
)


def _fin_body(a_ref, b_ref, o_ref):
    o_ref[...] = (a_ref[...] + b_ref[...])[:, :OUT]


_fin = pl.pallas_call(
    _fin_body,
    grid=(N // _B,),
    in_specs=[pl.BlockSpec((_B, LN), lambda i: (i, 0)),
              pl.BlockSpec((_B, LN), lambda i: (i, 0))],
    out_specs=pl.BlockSpec((_B, OUT), lambda i: (i, 0)),
    out_shape=jax.ShapeDtypeStruct((N, OUT), _f32),
)


def kernel(x, edge_index, edge_weight, W1, W2, W3):
    x2 = x.reshape(N, F * C)
    w1p = jnp.pad(W1.transpose(0, 2, 1).reshape(F * C, H1),
                  ((0, 0), (0, 2 * LN - H1)))
    pad = EP - E0
    src2 = jnp.concatenate(
        [edge_index[0], jnp.zeros((pad,), jnp.int32)]).reshape(EP // SEG, SEG)
    dst2 = jnp.concatenate(
        [edge_index[1], jnp.zeros((pad,), jnp.int32)]).reshape(EP // SEG, SEG)
    w1d = jnp.concatenate([edge_weight, jnp.zeros((pad,), _f32)])

    tlo, thi = _mm1(x2, w1p)
    hlo, hhi = _spmm12_relu(src2, dst2, w1d, tlo, thi)
    s2lo, s2hi = _spmm12_plain(src2, dst2, w1d, hlo, hhi)
    w2p = jnp.pad(W2, ((0, 2 * LN - H1), (0, 0)))
    w3p = jnp.pad(W3, ((0, 0), (0, LN - OUT)))
    g3 = _mm2(s2lo, s2hi, w2p, w3p)
    p0, p1 = _spmm3(src2, dst2, w1d, g3)
    return _fin(p0, p1)


# speedup vs baseline: 33.4759x; 1.3299x over previous
import functools

import jax
import jax.numpy as jnp
from jax import lax
from jax.experimental import pallas as pl
from jax.experimental.pallas import tpu as pltpu
from jax.experimental.pallas import tpu_sc as plsc

N = 100000
F, C = 128, 3
H1, H2, OUT = 20, 40, 2
E0 = 3200000
LN = 16
SEG = 128
CHUNK = 512
SUB = CHUNK // SEG
EP = 32 * CHUNK * 198
ROWS_L12 = EP // 16 // SEG
ROWS_L3 = EP // 32 // SEG
NCH_L12 = ROWS_L12 // SUB
NCH_L3 = ROWS_L3 // SUB
NP = 100096
OUT_PT = NP // 16

_f32 = jnp.float32


def _out_blocks():
    blks, o = [], 0
    while o < OUT_PT:
        n = min(CHUNK, OUT_PT - o)
        blks.append((o, n))
        o += n
    return blks


def _fill_rows(rows, n, val):
    def body(i, _):
        rows[0, i, :] = jnp.full((LN,), val, _f32)
        return 0
    lax.fori_loop(0, n, body, 0)


def _scale_rows(rows, wv, b):
    def body(i, _):
        w16 = wv[b, pl.ds(i * LN, LN)]
        e0 = i * LN
        for k in range(LN):
            rows[b, e0 + k, :] = rows[b, e0 + k, :] * w16[k]
        return 0
    lax.fori_loop(0, CHUNK // LN, body, 0)


def _relu_rows(rows, n):
    def body(i, _):
        rows[0, i, :] = jnp.maximum(rows[0, i, :], 0.0)
        return 0
    lax.fori_loop(0, n, body, 0, unroll=8)


def _spmm_core(src2, dst2, w1d, table, out, srcv, dstv, wv, rows, acc,
               lsem, gsems, ssems, tid, n_chunks, edge_row_base, relu):
    def drain(sem, b):
        pltpu.make_async_copy(table.at[pl.ds(0, CHUNK)], rows.at[b], sem).wait()

    def load_idx(chunk_idx, b):
        row0 = edge_row_base + chunk_idx * SUB
        cps = [pltpu.async_copy(src2.at[pl.ds(row0, SUB)], srcv.at[b], lsem),
               pltpu.async_copy(dst2.at[pl.ds(row0, SUB)], dstv.at[b], lsem),
               pltpu.async_copy(w1d.at[pl.ds(row0 * SEG, CHUNK)], wv.at[b],
                                lsem)]
        for cp in cps:
            cp.wait()

    def fire_gathers(b):
        for j in range(SUB):
            pltpu.async_copy(table.at[srcv.at[b, j]],
                             rows.at[b, pl.ds(j * SEG, SEG)], gsems[b])

    def fire_scatters(b):
        for j in range(SUB):
            pltpu.async_copy(rows.at[b, pl.ds(j * SEG, SEG)],
                             acc.at[dstv.at[b, j]], ssems[b], add=True)

    def stage(cur, k, drain_scatter):
        y = (k + 1) % 3
        if drain_scatter:
            drain(ssems[y], y)
        nxt = jnp.minimum(cur + 1, n_chunks - 1)
        load_idx(nxt, y)
        fire_gathers(y)
        drain(gsems[k], k)
        _scale_rows(rows, wv, k)
        fire_scatters(k)

    _fill_rows(rows, CHUNK, 0.0)
    ob = tid * OUT_PT
    for (o, n) in _out_blocks():
        pltpu.sync_copy(rows.at[0, pl.ds(0, n)], acc.at[pl.ds(ob + o, n)])
    plsc.subcore_barrier()

    load_idx(0, 0)
    fire_gathers(0)
    stage(0, 0, False)
    stage(1, 1, False)
    stage(2, 2, True)

    def triple(g, _):
        stage(3 * g, 0, True)
        stage(3 * g + 1, 1, True)
        stage(3 * g + 2, 2, True)
        return 0

    lax.fori_loop(1, n_chunks // 3, triple, 0)
    drain(gsems[0], 0)
    drain(ssems[(n_chunks - 2) % 3], 0)
    drain(ssems[(n_chunks - 1) % 3], 1)
    plsc.subcore_barrier()

    if relu:
        for (o, n) in _out_blocks():
            pltpu.sync_copy(acc.at[pl.ds(ob + o, n)], rows.at[0, pl.ds(0, n)])
            _relu_rows(rows, n)
            pltpu.sync_copy(rows.at[0, pl.ds(0, n)], out.at[pl.ds(ob + o, n)])
    else:
        pltpu.sync_copy(acc.at[pl.ds(ob, OUT_PT)], out.at[pl.ds(ob, OUT_PT)])


def _sc_scratch():
    return [
        pltpu.VMEM((3, SUB, SEG), jnp.int32),
        pltpu.VMEM((3, SUB, SEG), jnp.int32),
        pltpu.VMEM((3, CHUNK), _f32),
        pltpu.VMEM((3, CHUNK, LN), _f32),
        pltpu.VMEM_SHARED((NP, LN), _f32),
        pltpu.SemaphoreType.DMA,
        pltpu.SemaphoreType.DMA,
        pltpu.SemaphoreType.DMA,
        pltpu.SemaphoreType.DMA,
        pltpu.SemaphoreType.DMA,
        pltpu.SemaphoreType.DMA,
        pltpu.SemaphoreType.DMA,
    ]


def _make_spmm12(relu):
    @functools.partial(
        pl.kernel,
        mesh=plsc.VectorSubcoreMesh(core_axis_name="c", subcore_axis_name="s"),
        out_type=[jax.ShapeDtypeStruct((NP, LN), _f32),
                  jax.ShapeDtypeStruct((NP, LN), _f32)],
        scratch_types=_sc_scratch(),
        compiler_params=pltpu.CompilerParams(use_tc_tiling_on_sc=False),
    )
    def k(src2, dst2, w1d, tlo, thi, out_lo, out_hi,
          srcv, dstv, wv, rows, acc, lsem, g0, g1, g2, s0, s1, s2):
        c = lax.axis_index("c")
        s = lax.axis_index("s")
        base = s * ROWS_L12

        @pl.when(c == 0)
        def _():
            _spmm_core(src2, dst2, w1d, tlo, out_lo, srcv, dstv, wv, rows,
                       acc, lsem, (g0, g1, g2), (s0, s1, s2),
                       s, NCH_L12, base, relu)

        @pl.when(c == 1)
        def _():
            _spmm_core(src2, dst2, w1d, thi, out_hi, srcv, dstv, wv, rows,
                       acc, lsem, (g0, g1, g2), (s0, s1, s2),
                       s, NCH_L12, base, relu)

    return k


def _make_spmm3():
    @functools.partial(
        pl.kernel,
        mesh=plsc.VectorSubcoreMesh(core_axis_name="c", subcore_axis_name="s"),
        out_type=[jax.ShapeDtypeStruct((NP, LN), _f32),
                  jax.ShapeDtypeStruct((NP, LN), _f32)],
        scratch_types=_sc_scratch(),
        compiler_params=pltpu.CompilerParams(use_tc_tiling_on_sc=False),
    )
    def k(src2, dst2, w1d, tbl, p0, p1, srcv, dstv, wv, rows, acc,
          lsem, g0, g1, g2, s0, s1, s2):
        c = lax.axis_index("c")
        s = lax.axis_index("s")
        base = (c * 16 + s) * ROWS_L3

        @pl.when(c == 0)
        def _():
            _spmm_core(src2, dst2, w1d, tbl, p0, srcv, dstv, wv, rows,
                       acc, lsem, (g0, g1, g2), (s0, s1, s2),
                       s, NCH_L3, base, False)

        @pl.when(c == 1)
        def _():
            _spmm_core(src2, dst2, w1d, tbl, p1, srcv, dstv, wv, rows,
                       acc, lsem, (g0, g1, g2), (s0, s1, s2),
                       s, NCH_L3, base, False)

    return k


_spmm12_relu = _make_spmm12(True)
_spmm12_plain = _make_spmm12(False)
_spmm3 = _make_spmm3()

_B = 1000


def _mm1_body(x_ref, w_ref, lo_ref, hi_ref):
    r = jnp.dot(x_ref[...], w_ref[...], preferred_element_type=_f32,
                precision=lax.Precision.HIGHEST)
    lo_ref[...] = r[:, :LN]
    hi_ref[...] = r[:, LN:]


_mm1 = pl.pallas_call(
    _mm1_body,
    grid=(N // _B,),
    in_specs=[pl.BlockSpec((_B, F * C), lambda i: (i, 0)),
              pl.BlockSpec((F * C, 2 * LN), lambda i: (0, 0))],
    out_specs=[pl.BlockSpec((_B, LN), lambda i: (i, 0)),
               pl.BlockSpec((_B, LN), lambda i: (i, 0))],
    out_shape=[jax.ShapeDtypeStruct((N, LN), _f32),
               jax.ShapeDtypeStruct((N, LN), _f32)],
)


def _mm2_body(lo_ref, hi_ref, w2_ref, w3_ref, g_ref):
    cat = jnp.concatenate([lo_ref[...], hi_ref[...]], axis=1)
    t = jnp.maximum(jnp.dot(cat, w2_ref[...], preferred_element_type=_f32,
                            precision=lax.Precision.HIGHEST), 0.0)
    g_ref[...] = jnp.dot(t, w3_ref[...], preferred_element_type=_f32,
                         precision=lax.Precision.HIGHEST)


_mm2 = pl.pallas_call(
    _mm2_body,
    grid=(N // _B,),
    in_specs=[pl.BlockSpec((_B, LN), lambda i: (i, 0)),
              pl.BlockSpec((_B, LN), lambda i: (i, 0)),
              pl.BlockSpec((2 * LN, H2), lambda i: (0, 0)),
              pl.BlockSpec((H2, LN), lambda i: (0, 0))],
    out_specs=pl.BlockSpec((_B, LN), lambda i: (i, 0)),
    out_shape=jax.ShapeDtypeStruct((N, LN), _f32),
)


def _fin_body(a_ref, b_ref, o_ref):
    o_ref[...] = (a_ref[...] + b_ref[...])[:, :OUT]


_fin = pl.pallas_call(
    _fin_body,
    grid=(N // _B,),
    in_specs=[pl.BlockSpec((_B, LN), lambda i: (i, 0)),
              pl.BlockSpec((_B, LN), lambda i: (i, 0))],
    out_specs=pl.BlockSpec((_B, OUT), lambda i: (i, 0)),
    out_shape=jax.ShapeDtypeStruct((N, OUT), _f32),
)


def kernel(x, edge_index, edge_weight, W1, W2, W3):
    x2 = x.reshape(N, F * C)
    w1p = jnp.pad(W1.transpose(0, 2, 1).reshape(F * C, H1),
                  ((0, 0), (0, 2 * LN - H1)))
    pad = EP - E0
    src2 = jnp.concatenate(
        [edge_index[0], jnp.zeros((pad,), jnp.int32)]).reshape(EP // SEG, SEG)
    dst2 = jnp.concatenate(
        [edge_index[1], jnp.zeros((pad,), jnp.int32)]).reshape(EP // SEG, SEG)
    w1d = jnp.concatenate([edge_weight, jnp.zeros((pad,), _f32)])

    tlo, thi = _mm1(x2, w1p)
    hlo, hhi = _spmm12_relu(src2, dst2, w1d, tlo, thi)
    s2lo, s2hi = _spmm12_plain(src2, dst2, w1d, hlo, hhi)
    w2p = jnp.pad(W2, ((0, 2 * LN - H1), (0, 0)))
    w3p = jnp.pad(W3, ((0, 0), (0, LN - OUT)))
    g3 = _mm2(s2lo, s2hi, w2p, w3p)
    p0, p1 = _spmm3(src2, dst2, w1d, g3)
    return _fin(p0, p1)


# speedup vs baseline: 33.4930x; 1.0005x over previous
import functools

import jax
import jax.numpy as jnp
from jax import lax
from jax.experimental import pallas as pl
from jax.experimental.pallas import tpu as pltpu
from jax.experimental.pallas import tpu_sc as plsc

N = 100000
F, C = 128, 3
H1, H2, OUT = 20, 40, 2
E0 = 3200000
LN = 16
SEG = 128
CHUNK = 512
SUB = CHUNK // SEG
EP = 32 * CHUNK * 198
ROWS_L12 = EP // 16 // SEG
ROWS_L3 = EP // 32 // SEG
NCH_L12 = ROWS_L12 // SUB
NCH_L3 = ROWS_L3 // SUB
NP = 100096
OUT_PT = NP // 16

_f32 = jnp.float32


def _out_blocks():
    blks, o = [], 0
    while o < OUT_PT:
        n = min(CHUNK, OUT_PT - o)
        blks.append((o, n))
        o += n
    return blks


def _fill_rows(rows, n, val):
    def body(i, _):
        rows[0, i, :] = jnp.full((LN,), val, _f32)
        return 0
    lax.fori_loop(0, n, body, 0)


def _scale_rows(rows, wv, b):
    def body(i, _):
        w16 = wv[b, pl.ds(i * LN, LN)]
        e0 = i * LN
        for k in range(LN):
            rows[b, e0 + k, :] = rows[b, e0 + k, :] * w16[k]
        return 0
    lax.fori_loop(0, CHUNK // LN, body, 0)


def _relu_rows(rows, n):
    def body(i, _):
        rows[0, i, :] = jnp.maximum(rows[0, i, :], 0.0)
        return 0
    lax.fori_loop(0, n, body, 0, unroll=8)


def _spmm_core(src2, dst2, w1d, table, out, srcv, dstv, wv, rows, acc,
               lsem, gsems, ssems, tid, n_chunks, edge_row_base, relu):
    def drain(sem, b):
        pltpu.make_async_copy(table.at[pl.ds(0, CHUNK)], rows.at[b], sem).wait()

    def load_idx(chunk_idx, b):
        row0 = edge_row_base + chunk_idx * SUB
        cs = pltpu.async_copy(src2.at[pl.ds(row0, SUB)], srcv.at[b], lsem)
        cd = pltpu.async_copy(dst2.at[pl.ds(row0, SUB)], dstv.at[b], lsem)
        cw = pltpu.async_copy(w1d.at[pl.ds(row0 * SEG, CHUNK)], wv.at[b], lsem)
        return cs, cd, cw

    def fire_gathers(b):
        for j in range(SUB):
            pltpu.async_copy(table.at[srcv.at[b, j]],
                             rows.at[b, pl.ds(j * SEG, SEG)], gsems[b])

    def fire_scatters(b):
        for j in range(SUB):
            pltpu.async_copy(rows.at[b, pl.ds(j * SEG, SEG)],
                             acc.at[dstv.at[b, j]], ssems[b], add=True)

    def stage(cur, k, drain_scatter):
        y = (k + 1) % 3
        if drain_scatter:
            drain(ssems[y], y)
        nxt = jnp.minimum(cur + 1, n_chunks - 1)
        cs, cd, cw = load_idx(nxt, y)
        cs.wait()
        fire_gathers(y)
        cd.wait()
        cw.wait()
        drain(gsems[k], k)
        _scale_rows(rows, wv, k)
        fire_scatters(k)

    _fill_rows(rows, CHUNK, 0.0)
    ob = tid * OUT_PT
    for (o, n) in _out_blocks():
        pltpu.sync_copy(rows.at[0, pl.ds(0, n)], acc.at[pl.ds(ob + o, n)])
    plsc.subcore_barrier()

    cs, cd, cw = load_idx(0, 0)
    cs.wait()
    fire_gathers(0)
    cd.wait()
    cw.wait()
    stage(0, 0, False)
    stage(1, 1, False)
    stage(2, 2, True)

    def triple(g, _):
        stage(3 * g, 0, True)
        stage(3 * g + 1, 1, True)
        stage(3 * g + 2, 2, True)
        return 0

    lax.fori_loop(1, n_chunks // 3, triple, 0)
    drain(gsems[0], 0)
    drain(ssems[(n_chunks - 2) % 3], 0)
    drain(ssems[(n_chunks - 1) % 3], 1)
    plsc.subcore_barrier()

    if relu:
        for (o, n) in _out_blocks():
            pltpu.sync_copy(acc.at[pl.ds(ob + o, n)], rows.at[0, pl.ds(0, n)])
            _relu_rows(rows, n)
            pltpu.sync_copy(rows.at[0, pl.ds(0, n)], out.at[pl.ds(ob + o, n)])
    else:
        pltpu.sync_copy(acc.at[pl.ds(ob, OUT_PT)], out.at[pl.ds(ob, OUT_PT)])


def _sc_scratch():
    return [
        pltpu.VMEM((3, SUB, SEG), jnp.int32),
        pltpu.VMEM((3, SUB, SEG), jnp.int32),
        pltpu.VMEM((3, CHUNK), _f32),
        pltpu.VMEM((3, CHUNK, LN), _f32),
        pltpu.VMEM_SHARED((NP, LN), _f32),
        pltpu.SemaphoreType.DMA,
        pltpu.SemaphoreType.DMA,
        pltpu.SemaphoreType.DMA,
        pltpu.SemaphoreType.DMA,
        pltpu.SemaphoreType.DMA,
        pltpu.SemaphoreType.DMA,
        pltpu.SemaphoreType.DMA,
    ]


_MESH = dict(core_axis_name="c", subcore_axis_name="s")


def _make_spmm12():
    @functools.partial(
        pl.kernel,
        mesh=plsc.VectorSubcoreMesh(**_MESH),
        out_type=[jax.ShapeDtypeStruct((2, NP, LN), _f32),
                  jax.ShapeDtypeStruct((2, NP, LN), _f32)],
        scratch_types=_sc_scratch(),
        compiler_params=pltpu.CompilerParams(use_tc_tiling_on_sc=False),
    )
    def k(src2, dst2, w1d, t, h, s2,
          srcv, dstv, wv, rows, acc, lsem, g0, g1, g2, s0, s1, s2s):
        c = lax.axis_index("c")
        s = lax.axis_index("s")
        base = s * ROWS_L12
        gsems = (g0, g1, g2)
        ssems = (s0, s1, s2s)
        _spmm_core(src2, dst2, w1d, t.at[c], h.at[c], srcv, dstv, wv, rows,
                   acc, lsem, gsems, ssems, s, NCH_L12, base, True)
        _spmm_core(src2, dst2, w1d, h.at[c], s2.at[c], srcv, dstv, wv, rows,
                   acc, lsem, gsems, ssems, s, NCH_L12, base, False)

    return k


def _make_spmm3():
    @functools.partial(
        pl.kernel,
        mesh=plsc.VectorSubcoreMesh(**_MESH),
        out_type=jax.ShapeDtypeStruct((2, NP, LN), _f32),
        scratch_types=_sc_scratch(),
        compiler_params=pltpu.CompilerParams(use_tc_tiling_on_sc=False),
    )
    def k(src2, dst2, w1d, tbl, p,
          srcv, dstv, wv, rows, acc, lsem, g0, g1, g2, s0, s1, s2s):
        c = lax.axis_index("c")
        s = lax.axis_index("s")
        base = (c * 16 + s) * ROWS_L3
        _spmm_core(src2, dst2, w1d, tbl, p.at[c], srcv, dstv, wv, rows,
                   acc, lsem, (g0, g1, g2), (s0, s1, s2s),
                   s, NCH_L3, base, False)

    return k


_spmm12 = _make_spmm12()
_spmm3 = _make_spmm3()

_B = 1000


def _mm1_body(x_ref, w_ref, t_ref):
    r = jnp.dot(x_ref[...], w_ref[...], preferred_element_type=_f32,
                precision=lax.Precision.HIGHEST)
    t_ref[0] = r[:, :LN]
    t_ref[1] = r[:, LN:]


_mm1 = pl.pallas_call(
    _mm1_body,
    grid=(N // _B,),
    in_specs=[pl.BlockSpec((_B, F * C), lambda i: (i, 0)),
              pl.BlockSpec((F * C, 2 * LN), lambda i: (0, 0))],
    out_specs=pl.BlockSpec((2, _B, LN), lambda i: (0, i, 0)),
    out_shape=jax.ShapeDtypeStruct((2, NP, LN), _f32),
)


def _mm2_body(s2_ref, w2_ref, w3_ref, g_ref):
    cat = jnp.concatenate([s2_ref[0], s2_ref[1]], axis=1)
    t = jnp.maximum(jnp.dot(cat, w2_ref[...], preferred_element_type=_f32,
                            precision=lax.Precision.HIGHEST), 0.0)
    g_ref[...] = jnp.dot(t, w3_ref[...], preferred_element_type=_f32,
                         precision=lax.Precision.HIGHEST)


_mm2 = pl.pallas_call(
    _mm2_body,
    grid=(N // _B,),
    in_specs=[pl.BlockSpec((2, _B, LN), lambda i: (0, i, 0)),
              pl.BlockSpec((2 * LN, H2), lambda i: (0, 0)),
              pl.BlockSpec((H2, LN), lambda i: (0, 0))],
    out_specs=pl.BlockSpec((_B, LN), lambda i: (i, 0)),
    out_shape=jax.ShapeDtypeStruct((N, LN), _f32),
)


def _fin_body(p_ref, o_ref):
    o_ref[...] = (p_ref[0] + p_ref[1])[:, :OUT]


_fin = pl.pallas_call(
    _fin_body,
    grid=(N // _B,),
    in_specs=[pl.BlockSpec((2, _B, LN), lambda i: (0, i, 0))],
    out_specs=pl.BlockSpec((_B, OUT), lambda i: (i, 0)),
    out_shape=jax.ShapeDtypeStruct((N, OUT), _f32),
)


def kernel(x, edge_index, edge_weight, W1, W2, W3):
    x2 = x.reshape(N, F * C)
    w1p = jnp.pad(W1.transpose(0, 2, 1).reshape(F * C, H1),
                  ((0, 0), (0, 2 * LN - H1)))
    pad = EP - E0
    src2 = jnp.concatenate(
        [edge_index[0], jnp.zeros((pad,), jnp.int32)]).reshape(EP // SEG, SEG)
    dst2 = jnp.concatenate(
        [edge_index[1], jnp.zeros((pad,), jnp.int32)]).reshape(EP // SEG, SEG)
    w1d = jnp.concatenate([edge_weight, jnp.zeros((pad,), _f32)])

    t = _mm1(x2, w1p)
    h, s2 = _spmm12(src2, dst2, w1d, t)
    w2p = jnp.pad(W2, ((0, 2 * LN - H1), (0, 0)))
    w3p = jnp.pad(W3, ((0, 0), (0, LN - OUT)))
    g3 = _mm2(s2, w2p, w3p)
    p = _spmm3(src2, dst2, w1d, g3)
    return _fin(p)


# speedup vs baseline: 37.0953x; 1.1076x over previous
import functools

import jax
import jax.numpy as jnp
from jax import lax
from jax.experimental import pallas as pl
from jax.experimental.pallas import tpu as pltpu
from jax.experimental.pallas import tpu_sc as plsc

N = 100000
F, C = 128, 3
H1, H2, OUT = 20, 40, 2
E0 = 3200000
LN = 16
SEG = 128
CHUNK = 512
SUB = CHUNK // SEG
EP = 32 * CHUNK * 198
ROWS_L12 = EP // 16 // SEG
ROWS_L3 = EP // 32 // SEG
NCH_L12 = ROWS_L12 // SUB
NCH_L3 = ROWS_L3 // SUB
NP = 100096
OUT_PT = NP // 16

_f32 = jnp.float32


def _out_blocks():
    blks, o = [], 0
    while o < OUT_PT:
        n = min(CHUNK, OUT_PT - o)
        blks.append((o, n))
        o += n
    return blks


def _fill_rows(rows, n, val):
    def body(i, _):
        rows[0, i, :] = jnp.full((LN,), val, _f32)
        return 0
    lax.fori_loop(0, n, body, 0)


def _scale_rows(rows, wv, b):
    def body(i, _):
        w16 = wv[b, pl.ds(i * LN, LN)]
        e0 = i * LN
        for k in range(LN):
            rows[b, e0 + k, :] = rows[b, e0 + k, :] * w16[k]
        return 0
    lax.fori_loop(0, CHUNK // LN, body, 0)


def _relu_rows(rows, n):
    def body(i, _):
        rows[0, i, :] = jnp.maximum(rows[0, i, :], 0.0)
        return 0
    lax.fori_loop(0, n, body, 0, unroll=8)


def _spmm_core(src2, dst2, w1d, table, out, srcv, dstv, wv, rows, acc,
               lsem, gsems, ssems, tid, n_chunks, edge_row_base, relu):
    def drain(sem, b):
        pltpu.make_async_copy(table.at[pl.ds(0, CHUNK)], rows.at[b], sem).wait()

    def load_idx(chunk_idx, b):
        row0 = edge_row_base + chunk_idx * SUB
        cs = pltpu.async_copy(src2.at[pl.ds(row0, SUB)], srcv.at[b], lsem)
        cd = pltpu.async_copy(dst2.at[pl.ds(row0, SUB)], dstv.at[b], lsem)
        cw = pltpu.async_copy(w1d.at[pl.ds(row0 * SEG, CHUNK)], wv.at[b], lsem)
        return cs, cd, cw

    def fire_gathers(b):
        for j in range(SUB):
            pltpu.async_copy(table.at[srcv.at[b, j]],
                             rows.at[b, pl.ds(j * SEG, SEG)], gsems[b])

    def fire_scatters(b):
        for j in range(SUB):
            pltpu.async_copy(rows.at[b, pl.ds(j * SEG, SEG)],
                             acc.at[dstv.at[b, j]], ssems[b], add=True)

    def stage(cur, k, drain_scatter):
        y = (k + 1) % 3
        if drain_scatter:
            drain(ssems[y], y)
        nxt = jnp.minimum(cur + 1, n_chunks - 1)
        cs, cd, cw = load_idx(nxt, y)
        cs.wait()
        fire_gathers(y)
        cd.wait()
        cw.wait()
        drain(gsems[k], k)
        _scale_rows(rows, wv, k)
        fire_scatters(k)

    _fill_rows(rows, CHUNK, 0.0)
    ob = tid * OUT_PT
    for (o, n) in _out_blocks():
        pltpu.sync_copy(rows.at[0, pl.ds(0, n)], acc.at[pl.ds(ob + o, n)])
    plsc.subcore_barrier()

    cs, cd, cw = load_idx(0, 0)
    cs.wait()
    fire_gathers(0)
    cd.wait()
    cw.wait()
    stage(0, 0, False)
    stage(1, 1, False)
    stage(2, 2, True)

    def triple(g, _):
        stage(3 * g, 0, True)
        stage(3 * g + 1, 1, True)
        stage(3 * g + 2, 2, True)
        return 0

    lax.fori_loop(1, n_chunks // 3, triple, 0)
    drain(gsems[0], 0)
    drain(ssems[(n_chunks - 2) % 3], 0)
    drain(ssems[(n_chunks - 1) % 3], 1)
    plsc.subcore_barrier()

    if relu:
        for (o, n) in _out_blocks():
            pltpu.sync_copy(acc.at[pl.ds(ob + o, n)], rows.at[0, pl.ds(0, n)])
            _relu_rows(rows, n)
            pltpu.sync_copy(rows.at[0, pl.ds(0, n)], out.at[pl.ds(ob + o, n)])
    else:
        pltpu.sync_copy(acc.at[pl.ds(ob, OUT_PT)], out.at[pl.ds(ob, OUT_PT)])


def _sc_scratch():
    return [
        pltpu.VMEM((3, SUB, SEG), jnp.int32),
        pltpu.VMEM((3, SUB, SEG), jnp.int32),
        pltpu.VMEM((3, CHUNK), _f32),
        pltpu.VMEM((3, CHUNK, LN), _f32),
        pltpu.VMEM_SHARED((NP, LN), _f32),
        pltpu.SemaphoreType.DMA,
        pltpu.SemaphoreType.DMA,
        pltpu.SemaphoreType.DMA,
        pltpu.SemaphoreType.DMA,
        pltpu.SemaphoreType.DMA,
        pltpu.SemaphoreType.DMA,
        pltpu.SemaphoreType.DMA,
    ]


_MESH = dict(core_axis_name="c", subcore_axis_name="s")


def _make_spmm12():
    @functools.partial(
        pl.kernel,
        mesh=plsc.VectorSubcoreMesh(**_MESH),
        out_type=[jax.ShapeDtypeStruct((2, NP, LN), _f32),
                  jax.ShapeDtypeStruct((2, NP, LN), _f32)],
        scratch_types=_sc_scratch(),
        compiler_params=pltpu.CompilerParams(use_tc_tiling_on_sc=False),
    )
    def k(src2, dst2, w1d, t, h, s2,
          srcv, dstv, wv, rows, acc, lsem, g0, g1, g2, s0, s1, s2s):
        c = lax.axis_index("c")
        s = lax.axis_index("s")
        base = s * ROWS_L12
        gsems = (g0, g1, g2)
        ssems = (s0, s1, s2s)
        _spmm_core(src2, dst2, w1d, t.at[c], h.at[c], srcv, dstv, wv, rows,
                   acc, lsem, gsems, ssems, s, NCH_L12, base, True)
        _spmm_core(src2, dst2, w1d, h.at[c], s2.at[c], srcv, dstv, wv, rows,
                   acc, lsem, gsems, ssems, s, NCH_L12, base, False)

    return k


def _make_spmm3():
    @functools.partial(
        pl.kernel,
        mesh=plsc.VectorSubcoreMesh(**_MESH),
        out_type=jax.ShapeDtypeStruct((2, NP, LN), _f32),
        scratch_types=_sc_scratch(),
        compiler_params=pltpu.CompilerParams(use_tc_tiling_on_sc=False),
    )
    def k(src2, dst2, w1d, tbl, p,
          srcv, dstv, wv, rows, acc, lsem, g0, g1, g2, s0, s1, s2s):
        c = lax.axis_index("c")
        s = lax.axis_index("s")
        base = (c * 16 + s) * ROWS_L3
        _spmm_core(src2, dst2, w1d, tbl, p.at[c], srcv, dstv, wv, rows,
                   acc, lsem, (g0, g1, g2), (s0, s1, s2s),
                   s, NCH_L3, base, False)

    return k


_spmm12 = _make_spmm12()
_spmm3 = _make_spmm3()

_B1 = 2000
_B2 = 4000


def _mm1_body(x_ref, w_ref, t_ref):
    r = jnp.dot(x_ref[...], w_ref[...], preferred_element_type=_f32)
    t_ref[0] = r[:, :LN]
    t_ref[1] = r[:, LN:]


_mm1 = pl.pallas_call(
    _mm1_body,
    grid=(N // _B1,),
    in_specs=[pl.BlockSpec((_B1, F * C), lambda i: (i, 0)),
              pl.BlockSpec((F * C, 2 * LN), lambda i: (0, 0))],
    out_specs=pl.BlockSpec((2, _B1, LN), lambda i: (0, i, 0)),
    out_shape=jax.ShapeDtypeStruct((2, NP, LN), _f32),
)


def _mm2_body(s2_ref, w2_ref, w3_ref, g_ref):
    cat = jnp.concatenate([s2_ref[0], s2_ref[1]], axis=1)
    t = jnp.maximum(jnp.dot(cat, w2_ref[...], preferred_element_type=_f32,
                            precision=lax.Precision.HIGHEST), 0.0)
    g_ref[...] = jnp.dot(t, w3_ref[...], preferred_element_type=_f32,
                         precision=lax.Precision.HIGHEST)


_mm2 = pl.pallas_call(
    _mm2_body,
    grid=(N // _B2,),
    in_specs=[pl.BlockSpec((2, _B2, LN), lambda i: (0, i, 0)),
              pl.BlockSpec((2 * LN, H2), lambda i: (0, 0)),
              pl.BlockSpec((H2, LN), lambda i: (0, 0))],
    out_specs=pl.BlockSpec((_B2, LN), lambda i: (i, 0)),
    out_shape=jax.ShapeDtypeStruct((N, LN), _f32),
)


def _fin_body(p_ref, o_ref):
    o_ref[...] = (p_ref[0] + p_ref[1])[:, :OUT]


_fin = pl.pallas_call(
    _fin_body,
    grid=(N // _B2,),
    in_specs=[pl.BlockSpec((2, _B2, LN), lambda i: (0, i, 0))],
    out_specs=pl.BlockSpec((_B2, OUT), lambda i: (i, 0)),
    out_shape=jax.ShapeDtypeStruct((N, OUT), _f32),
)


def kernel(x, edge_index, edge_weight, W1, W2, W3):
    x2 = x.astype(jnp.bfloat16).reshape(N, F * C)
    w1p = jnp.pad(W1.transpose(0, 2, 1).reshape(F * C, H1),
                  ((0, 0), (0, 2 * LN - H1))).astype(jnp.bfloat16)
    pad = EP - E0
    src2 = jnp.concatenate(
        [edge_index[0], jnp.zeros((pad,), jnp.int32)]).reshape(EP // SEG, SEG)
    dst2 = jnp.concatenate(
        [edge_index[1], jnp.zeros((pad,), jnp.int32)]).reshape(EP // SEG, SEG)
    w1d = jnp.concatenate([edge_weight, jnp.zeros((pad,), _f32)])

    t = _mm1(x2, w1p)
    h, s2 = _spmm12(src2, dst2, w1d, t)
    w2p = jnp.pad(W2, ((0, 2 * LN - H1), (0, 0)))
    w3p = jnp.pad(W3, ((0, 0), (0, LN - OUT)))
    g3 = _mm2(s2, w2p, w3p)
    p = _spmm3(src2, dst2, w1d, g3)
    return _fin(p)


# speedup vs baseline: 37.6474x; 1.0149x over previous
import functools

import jax
import jax.numpy as jnp
from jax import lax
from jax.experimental import pallas as pl
from jax.experimental.pallas import tpu as pltpu
from jax.experimental.pallas import tpu_sc as plsc

N = 100000
F, C = 128, 3
H1, H2, OUT = 20, 40, 2
E0 = 3200000
LN = 16
SEG = 128
CHUNK = 512
SUB = CHUNK // SEG
EP = 32 * CHUNK * 198
ROWS_L12 = EP // 16 // SEG
ROWS_L3 = EP // 32 // SEG
NCH_L12 = ROWS_L12 // SUB
NCH_L3 = ROWS_L3 // SUB
NP = 100096
OUT_PT = NP // 16

_f32 = jnp.float32


def _out_blocks():
    blks, o = [], 0
    while o < OUT_PT:
        n = min(CHUNK, OUT_PT - o)
        blks.append((o, n))
        o += n
    return blks


def _fill_rows(rows, n, val):
    def body(i, _):
        rows[0, i, :] = jnp.full((LN,), val, _f32)
        return 0
    lax.fori_loop(0, n, body, 0)


def _scale_rows(rows, wv, b):
    @plsc.parallel_loop(0, CHUNK // LN, unroll=2)
    def body(i):
        w16 = wv[b, pl.ds(i * LN, LN)]
        e0 = i * LN
        for k in range(LN):
            rows[b, e0 + k, :] = rows[b, e0 + k, :] * w16[k]


def _relu_rows(rows, n):
    def body(i, _):
        rows[0, i, :] = jnp.maximum(rows[0, i, :], 0.0)
        return 0
    lax.fori_loop(0, n, body, 0, unroll=8)


def _spmm_core(src2, dst2, w1d, table, out, srcv, dstv, wv, rows, acc,
               lsem, gsems, ssems, tid, n_chunks, edge_row_base, relu):
    def drain(sem, b):
        pltpu.make_async_copy(table.at[pl.ds(0, CHUNK)], rows.at[b], sem).wait()

    def load_idx(chunk_idx, b):
        row0 = edge_row_base + chunk_idx * SUB
        cs = pltpu.async_copy(src2.at[pl.ds(row0, SUB)], srcv.at[b], lsem)
        cd = pltpu.async_copy(dst2.at[pl.ds(row0, SUB)], dstv.at[b], lsem)
        cw = pltpu.async_copy(w1d.at[pl.ds(row0 * SEG, CHUNK)], wv.at[b], lsem)
        return cs, cd, cw

    def fire_gathers(b):
        for j in range(SUB):
            pltpu.async_copy(table.at[srcv.at[b, j]],
                             rows.at[b, pl.ds(j * SEG, SEG)], gsems[b])

    def fire_scatters(b):
        for j in range(SUB):
            pltpu.async_copy(rows.at[b, pl.ds(j * SEG, SEG)],
                             acc.at[dstv.at[b, j]], ssems[b], add=True)

    def stage(cur, k, drain_scatter):
        y = (k + 1) % 3
        if drain_scatter:
            drain(ssems[y], y)
        nxt = jnp.minimum(cur + 1, n_chunks - 1)
        cs, cd, cw = load_idx(nxt, y)
        cs.wait()
        fire_gathers(y)
        cd.wait()
        cw.wait()
        drain(gsems[k], k)
        _scale_rows(rows, wv, k)
        fire_scatters(k)

    _fill_rows(rows, CHUNK, 0.0)
    ob = tid * OUT_PT
    for (o, n) in _out_blocks():
        pltpu.sync_copy(rows.at[0, pl.ds(0, n)], acc.at[pl.ds(ob + o, n)])
    plsc.subcore_barrier()

    cs, cd, cw = load_idx(0, 0)
    cs.wait()
    fire_gathers(0)
    cd.wait()
    cw.wait()
    stage(0, 0, False)
    stage(1, 1, False)
    stage(2, 2, True)

    def triple(g, _):
        stage(3 * g, 0, True)
        stage(3 * g + 1, 1, True)
        stage(3 * g + 2, 2, True)
        return 0

    lax.fori_loop(1, n_chunks // 3, triple, 0)
    drain(gsems[0], 0)
    drain(ssems[(n_chunks - 2) % 3], 0)
    drain(ssems[(n_chunks - 1) % 3], 1)
    plsc.subcore_barrier()

    if relu:
        for (o, n) in _out_blocks():
            pltpu.sync_copy(acc.at[pl.ds(ob + o, n)], rows.at[0, pl.ds(0, n)])
            _relu_rows(rows, n)
            pltpu.sync_copy(rows.at[0, pl.ds(0, n)], out.at[pl.ds(ob + o, n)])
    else:
        pltpu.sync_copy(acc.at[pl.ds(ob, OUT_PT)], out.at[pl.ds(ob, OUT_PT)])


def _sc_scratch():
    return [
        pltpu.VMEM((3, SUB, SEG), jnp.int32),
        pltpu.VMEM((3, SUB, SEG), jnp.int32),
        pltpu.VMEM((3, CHUNK), _f32),
        pltpu.VMEM((3, CHUNK, LN), _f32),
        pltpu.VMEM_SHARED((NP, LN), _f32),
        pltpu.SemaphoreType.DMA,
        pltpu.SemaphoreType.DMA,
        pltpu.SemaphoreType.DMA,
        pltpu.SemaphoreType.DMA,
        pltpu.SemaphoreType.DMA,
        pltpu.SemaphoreType.DMA,
        pltpu.SemaphoreType.DMA,
    ]


_MESH = dict(core_axis_name="c", subcore_axis_name="s")


def _make_spmm12():
    @functools.partial(
        pl.kernel,
        mesh=plsc.VectorSubcoreMesh(**_MESH),
        out_type=[jax.ShapeDtypeStruct((2, NP, LN), _f32),
                  jax.ShapeDtypeStruct((2, NP, LN), _f32)],
        scratch_types=_sc_scratch(),
        compiler_params=pltpu.CompilerParams(use_tc_tiling_on_sc=False),
    )
    def k(src2, dst2, w1d, t, h, s2,
          srcv, dstv, wv, rows, acc, lsem, g0, g1, g2, s0, s1, s2s):
        c = lax.axis_index("c")
        s = lax.axis_index("s")
        base = s * ROWS_L12
        gsems = (g0, g1, g2)
        ssems = (s0, s1, s2s)
        _spmm_core(src2, dst2, w1d, t.at[c], h.at[c], srcv, dstv, wv, rows,
                   acc, lsem, gsems, ssems, s, NCH_L12, base, True)
        _spmm_core(src2, dst2, w1d, h.at[c], s2.at[c], srcv, dstv, wv, rows,
                   acc, lsem, gsems, ssems, s, NCH_L12, base, False)

    return k


def _make_spmm3():
    @functools.partial(
        pl.kernel,
        mesh=plsc.VectorSubcoreMesh(**_MESH),
        out_type=jax.ShapeDtypeStruct((2, NP, LN), _f32),
        scratch_types=_sc_scratch(),
        compiler_params=pltpu.CompilerParams(use_tc_tiling_on_sc=False),
    )
    def k(src2, dst2, w1d, tbl, p,
          srcv, dstv, wv, rows, acc, lsem, g0, g1, g2, s0, s1, s2s):
        c = lax.axis_index("c")
        s = lax.axis_index("s")
        base = (c * 16 + s) * ROWS_L3
        _spmm_core(src2, dst2, w1d, tbl, p.at[c], srcv, dstv, wv, rows,
                   acc, lsem, (g0, g1, g2), (s0, s1, s2s),
                   s, NCH_L3, base, False)

    return k


_spmm12 = _make_spmm12()
_spmm3 = _make_spmm3()

_B1 = 2000
_B2 = 4000


def _mm1_body(x_ref, w_ref, t_ref):
    r = jnp.dot(x_ref[...], w_ref[...], preferred_element_type=_f32)
    t_ref[0] = r[:, :LN]
    t_ref[1] = r[:, LN:]


_mm1 = pl.pallas_call(
    _mm1_body,
    grid=(N // _B1,),
    in_specs=[pl.BlockSpec((_B1, F * C), lambda i: (i, 0)),
              pl.BlockSpec((F * C, 2 * LN), lambda i: (0, 0))],
    out_specs=pl.BlockSpec((2, _B1, LN), lambda i: (0, i, 0)),
    out_shape=jax.ShapeDtypeStruct((2, NP, LN), _f32),
)


def _mm2_body(s2_ref, w2_ref, w3_ref, g_ref):
    cat = jnp.concatenate([s2_ref[0], s2_ref[1]], axis=1)
    t = jnp.maximum(jnp.dot(cat, w2_ref[...], preferred_element_type=_f32,
                            precision=lax.Precision.HIGHEST), 0.0)
    g_ref[...] = jnp.dot(t, w3_ref[...], preferred_element_type=_f32,
                         precision=lax.Precision.HIGHEST)


_mm2 = pl.pallas_call(
    _mm2_body,
    grid=(N // _B2,),
    in_specs=[pl.BlockSpec((2, _B2, LN), lambda i: (0, i, 0)),
              pl.BlockSpec((2 * LN, H2), lambda i: (0, 0)),
              pl.BlockSpec((H2, LN), lambda i: (0, 0))],
    out_specs=pl.BlockSpec((_B2, LN), lambda i: (i, 0)),
    out_shape=jax.ShapeDtypeStruct((N, LN), _f32),
)


def _fin_body(p_ref, o_ref):
    o_ref[...] = (p_ref[0] + p_ref[1])[:, :OUT]


_fin = pl.pallas_call(
    _fin_body,
    grid=(N // _B2,),
    in_specs=[pl.BlockSpec((2, _B2, LN), lambda i: (0, i, 0))],
    out_specs=pl.BlockSpec((_B2, OUT), lambda i: (i, 0)),
    out_shape=jax.ShapeDtypeStruct((N, OUT), _f32),
)


def kernel(x, edge_index, edge_weight, W1, W2, W3):
    x2 = x.astype(jnp.bfloat16).reshape(N, F * C)
    w1p = jnp.pad(W1.transpose(0, 2, 1).reshape(F * C, H1),
                  ((0, 0), (0, 2 * LN - H1))).astype(jnp.bfloat16)
    pad = EP - E0
    src2 = jnp.concatenate(
        [edge_index[0], jnp.zeros((pad,), jnp.int32)]).reshape(EP // SEG, SEG)
    dst2 = jnp.concatenate(
        [edge_index[1], jnp.zeros((pad,), jnp.int32)]).reshape(EP // SEG, SEG)
    w1d = jnp.concatenate([edge_weight, jnp.zeros((pad,), _f32)])

    t = _mm1(x2, w1p)
    h, s2 = _spmm12(src2, dst2, w1d, t)
    w2p = jnp.pad(W2, ((0, 2 * LN - H1), (0, 0)))
    w3p = jnp.pad(W3, ((0, 0), (0, LN - OUT)))
    g3 = _mm2(s2, w2p, w3p)
    p = _spmm3(src2, dst2, w1d, g3)
    return _fin(p)


# speedup vs baseline: 43.5463x; 1.1567x over previous
import functools

import jax
import jax.numpy as jnp
from jax import lax
from jax.experimental import pallas as pl
from jax.experimental.pallas import tpu as pltpu
from jax.experimental.pallas import tpu_sc as plsc

N = 100000
F, C = 128, 3
H1, H2, OUT = 20, 40, 2
E0 = 3200000
LN = 16
SEG = 128
CHUNK = 512
SUB = CHUNK // SEG
EP = 32 * CHUNK * 198
ROWS_L12 = EP // 16 // SEG
ROWS_L3 = EP // 32 // SEG
NCH_L12 = ROWS_L12 // SUB
NCH_L3 = ROWS_L3 // SUB
NP = 100096
OUT_PT = NP // 16

_f32 = jnp.float32


def _out_blocks():
    blks, o = [], 0
    while o < OUT_PT:
        n = min(CHUNK, OUT_PT - o)
        blks.append((o, n))
        o += n
    return blks


def _fill_rows(rows, n, val):
    def body(i, _):
        rows[0, i, :] = jnp.full((LN,), val, _f32)
        return 0
    lax.fori_loop(0, n, body, 0)


def _scale_rows(rows, wv, b):
    @plsc.parallel_loop(0, CHUNK // LN, unroll=2)
    def body(i):
        w16 = wv[b, pl.ds(i * LN, LN)]
        e0 = i * LN
        for k in range(LN):
            rows[b, e0 + k, :] = rows[b, e0 + k, :] * w16[k]


def _relu_rows(rows, n):
    def body(i, _):
        rows[0, i, :] = jnp.maximum(rows[0, i, :], 0.0)
        return 0
    lax.fori_loop(0, n, body, 0, unroll=8)


def _spmm_core(src2, dst2, w1d, table, out, srcv, dstv, wv, rows, acc,
               lsem, gsems, ssems, tid, n_chunks, edge_row_base, relu):
    def drain(sem, b):
        pltpu.make_async_copy(table.at[pl.ds(0, CHUNK)], rows.at[b], sem).wait()

    def load_idx(chunk_idx, b):
        row0 = edge_row_base + chunk_idx * SUB
        cs = pltpu.async_copy(src2.at[pl.ds(row0, SUB)], srcv.at[b], lsem)
        cd = pltpu.async_copy(dst2.at[pl.ds(row0, SUB)], dstv.at[b], lsem)
        cw = pltpu.async_copy(w1d.at[pl.ds(row0 * SEG, CHUNK)], wv.at[b], lsem)
        return cs, cd, cw

    def fire_gathers(b):
        for j in range(SUB):
            pltpu.async_copy(table.at[srcv.at[b, j]],
                             rows.at[b, pl.ds(j * SEG, SEG)], gsems[b])

    def fire_scatters(b):
        for j in range(SUB):
            pltpu.async_copy(rows.at[b, pl.ds(j * SEG, SEG)],
                             acc.at[dstv.at[b, j]], ssems[b], add=True)

    def stage(cur, k, drain_scatter):
        y = (k + 1) % 3
        if drain_scatter:
            drain(ssems[y], y)
        nxt = jnp.minimum(cur + 1, n_chunks - 1)
        cs, cd, cw = load_idx(nxt, y)
        cs.wait()
        fire_gathers(y)
        cd.wait()
        cw.wait()
        drain(gsems[k], k)
        _scale_rows(rows, wv, k)
        fire_scatters(k)

    _fill_rows(rows, CHUNK, 0.0)
    ob = tid * OUT_PT
    for (o, n) in _out_blocks():
        pltpu.sync_copy(rows.at[0, pl.ds(0, n)], acc.at[pl.ds(ob + o, n)])
    plsc.subcore_barrier()

    cs, cd, cw = load_idx(0, 0)
    cs.wait()
    fire_gathers(0)
    cd.wait()
    cw.wait()
    stage(0, 0, False)
    stage(1, 1, False)
    stage(2, 2, True)

    def triple(g, _):
        stage(3 * g, 0, True)
        stage(3 * g + 1, 1, True)
        stage(3 * g + 2, 2, True)
        return 0

    lax.fori_loop(1, n_chunks // 3, triple, 0)
    drain(gsems[0], 0)
    drain(ssems[(n_chunks - 2) % 3], 0)
    drain(ssems[(n_chunks - 1) % 3], 1)
    plsc.subcore_barrier()

    if relu:
        for (o, n) in _out_blocks():
            pltpu.sync_copy(acc.at[pl.ds(ob + o, n)], rows.at[0, pl.ds(0, n)])
            _relu_rows(rows, n)
            pltpu.sync_copy(rows.at[0, pl.ds(0, n)], out.at[pl.ds(ob + o, n)])
    else:
        pltpu.sync_copy(acc.at[pl.ds(ob, OUT_PT)], out.at[pl.ds(ob, OUT_PT)])


def _sc_scratch():
    return [
        pltpu.VMEM((3, SUB, SEG), jnp.int32),
        pltpu.VMEM((3, SUB, SEG), jnp.int32),
        pltpu.VMEM((3, CHUNK), _f32),
        pltpu.VMEM((3, CHUNK, LN), _f32),
        pltpu.VMEM_SHARED((NP, LN), _f32),
        pltpu.SemaphoreType.DMA,
        pltpu.SemaphoreType.DMA,
        pltpu.SemaphoreType.DMA,
        pltpu.SemaphoreType.DMA,
        pltpu.SemaphoreType.DMA,
        pltpu.SemaphoreType.DMA,
        pltpu.SemaphoreType.DMA,
    ]


_MESH = dict(core_axis_name="c", subcore_axis_name="s")


def _make_spmm12():
    @functools.partial(
        pl.kernel,
        mesh=plsc.VectorSubcoreMesh(**_MESH),
        out_type=[jax.ShapeDtypeStruct((2, NP, LN), _f32),
                  jax.ShapeDtypeStruct((2, NP, LN), _f32)],
        scratch_types=_sc_scratch(),
        compiler_params=pltpu.CompilerParams(use_tc_tiling_on_sc=False),
    )
    def k(src2, dst2, w1d, t, h, s2,
          srcv, dstv, wv, rows, acc, lsem, g0, g1, g2, s0, s1, s2s):
        c = lax.axis_index("c")
        s = lax.axis_index("s")
        base = s * ROWS_L12
        gsems = (g0, g1, g2)
        ssems = (s0, s1, s2s)
        _spmm_core(src2, dst2, w1d, t.at[c], h.at[c], srcv, dstv, wv, rows,
                   acc, lsem, gsems, ssems, s, NCH_L12, base, True)
        _spmm_core(src2, dst2, w1d, h.at[c], s2.at[c], srcv, dstv, wv, rows,
                   acc, lsem, gsems, ssems, s, NCH_L12, base, False)

    return k


def _make_spmm3():
    @functools.partial(
        pl.kernel,
        mesh=plsc.VectorSubcoreMesh(**_MESH),
        out_type=jax.ShapeDtypeStruct((2, NP, LN), _f32),
        scratch_types=_sc_scratch(),
        compiler_params=pltpu.CompilerParams(use_tc_tiling_on_sc=False),
    )
    def k(src2, dst2, w1d, tbl, p,
          srcv, dstv, wv, rows, acc, lsem, g0, g1, g2, s0, s1, s2s):
        c = lax.axis_index("c")
        s = lax.axis_index("s")
        base = (c * 16 + s) * ROWS_L3
        _spmm_core(src2, dst2, w1d, tbl, p.at[c], srcv, dstv, wv, rows,
                   acc, lsem, (g0, g1, g2), (s0, s1, s2s),
                   s, NCH_L3, base, False)

    return k


_spmm12 = _make_spmm12()
_spmm3 = _make_spmm3()

_B1 = 2000
_B2 = 4000


def _mm1_body(x_ref, w_ref, t_ref):
    r = jnp.dot(x_ref[0], w_ref[0], preferred_element_type=_f32)
    r += jnp.dot(x_ref[1], w_ref[1], preferred_element_type=_f32)
    r += jnp.dot(x_ref[2], w_ref[2], preferred_element_type=_f32)
    t_ref[0] = r[:, :LN]
    t_ref[1] = r[:, LN:]


_mm1 = pl.pallas_call(
    _mm1_body,
    grid=(N // _B1,),
    in_specs=[pl.BlockSpec((C, _B1, F), lambda i: (0, i, 0)),
              pl.BlockSpec((C, F, 2 * LN), lambda i: (0, 0, 0))],
    out_specs=pl.BlockSpec((2, _B1, LN), lambda i: (0, i, 0)),
    out_shape=jax.ShapeDtypeStruct((2, NP, LN), _f32),
)


def _mm2_body(s2_ref, w2_ref, w3_ref, g_ref):
    cat = jnp.concatenate([s2_ref[0], s2_ref[1]], axis=1)
    t = jnp.maximum(jnp.dot(cat, w2_ref[...], preferred_element_type=_f32,
                            precision=lax.Precision.HIGHEST), 0.0)
    g_ref[...] = jnp.dot(t, w3_ref[...], preferred_element_type=_f32,
                         precision=lax.Precision.HIGHEST)


_mm2 = pl.pallas_call(
    _mm2_body,
    grid=(N // _B2,),
    in_specs=[pl.BlockSpec((2, _B2, LN), lambda i: (0, i, 0)),
              pl.BlockSpec((2 * LN, H2), lambda i: (0, 0)),
              pl.BlockSpec((H2, LN), lambda i: (0, 0))],
    out_specs=pl.BlockSpec((_B2, LN), lambda i: (i, 0)),
    out_shape=jax.ShapeDtypeStruct((N, LN), _f32),
)


def _fin_body(p_ref, o_ref):
    o_ref[...] = (p_ref[0] + p_ref[1])[:, :OUT]


_fin = pl.pallas_call(
    _fin_body,
    grid=(N // _B2,),
    in_specs=[pl.BlockSpec((2, _B2, LN), lambda i: (0, i, 0))],
    out_specs=pl.BlockSpec((_B2, OUT), lambda i: (i, 0)),
    out_shape=jax.ShapeDtypeStruct((N, OUT), _f32),
)


def kernel(x, edge_index, edge_weight, W1, W2, W3):
    x2 = x.astype(jnp.bfloat16).transpose(2, 0, 1)
    w1p = jnp.pad(W1.transpose(2, 0, 1),
                  ((0, 0), (0, 0), (0, 2 * LN - H1))).astype(jnp.bfloat16)
    pad = EP - E0
    src2 = jnp.concatenate(
        [edge_index[0], jnp.zeros((pad,), jnp.int32)]).reshape(EP // SEG, SEG)
    dst2 = jnp.concatenate(
        [edge_index[1], jnp.zeros((pad,), jnp.int32)]).reshape(EP // SEG, SEG)
    w1d = jnp.concatenate([edge_weight, jnp.zeros((pad,), _f32)])

    t = _mm1(x2, w1p)
    h, s2 = _spmm12(src2, dst2, w1d, t)
    w2p = jnp.pad(W2, ((0, 2 * LN - H1), (0, 0)))
    w3p = jnp.pad(W3, ((0, 0), (0, LN - OUT)))
    g3 = _mm2(s2, w2p, w3p)
    p = _spmm3(src2, dst2, w1d, g3)
    return _fin(p)


# speedup vs baseline: 46.7383x; 1.0733x over previous
import functools

import jax
import jax.numpy as jnp
from jax import lax
from jax.experimental import pallas as pl
from jax.experimental.pallas import tpu as pltpu
from jax.experimental.pallas import tpu_sc as plsc

N = 100000
F, C = 128, 3
H1, H2, OUT = 20, 40, 2
E0 = 3200000
LN = 16
SEG = 128
CHUNK = 512
SUB = CHUNK // SEG
EP = 32 * CHUNK * 198
ROWS_L12 = EP // 16 // SEG
ROWS_L3 = EP // 32 // SEG
NCH_L12 = ROWS_L12 // SUB
NCH_L3 = ROWS_L3 // SUB
NP = 100096
OUT_PT = NP // 16

_f32 = jnp.float32


def _out_blocks():
    blks, o = [], 0
    while o < OUT_PT:
        n = min(CHUNK, OUT_PT - o)
        blks.append((o, n))
        o += n
    return blks


def _fill_rows(rows, n, val):
    def body(i, _):
        rows[0, i, :] = jnp.full((LN,), val, _f32)
        return 0
    lax.fori_loop(0, n, body, 0)


def _scale_rows(rows, wv, b):
    @plsc.parallel_loop(0, CHUNK // LN, unroll=2)
    def body(i):
        w16 = wv[b, pl.ds(i * LN, LN)]
        e0 = i * LN
        for k in range(LN):
            rows[b, e0 + k, :] = rows[b, e0 + k, :] * w16[k]


def _relu_rows(rows, n):
    def body(i, _):
        rows[0, i, :] = jnp.maximum(rows[0, i, :], 0.0)
        return 0
    lax.fori_loop(0, n, body, 0, unroll=8)


def _spmm_core(src2, dst2, w1d, table, out, srcv, dstv, wv, rows, acc,
               lsem, gsems, ssems, tid, n_chunks, edge_row_base, relu):
    def drain(sem, b):
        pltpu.make_async_copy(table.at[pl.ds(0, CHUNK)], rows.at[b], sem).wait()

    def load_idx(chunk_idx, b):
        row0 = edge_row_base + chunk_idx * SUB
        cs = pltpu.async_copy(src2.at[pl.ds(row0, SUB)], srcv.at[b], lsem)
        cd = pltpu.async_copy(dst2.at[pl.ds(row0, SUB)], dstv.at[b], lsem)
        cw = pltpu.async_copy(w1d.at[pl.ds(row0 * SEG, CHUNK)], wv.at[b], lsem)
        return cs, cd, cw

    def fire_gathers(b):
        for j in range(SUB):
            pltpu.async_copy(table.at[srcv.at[b, j]],
                             rows.at[b, pl.ds(j * SEG, SEG)], gsems[b])

    def fire_scatters(b):
        for j in range(SUB):
            pltpu.async_copy(rows.at[b, pl.ds(j * SEG, SEG)],
                             acc.at[dstv.at[b, j]], ssems[b], add=True)

    def stage(cur, k, drain_scatter):
        y = (k + 1) % 3
        if drain_scatter:
            drain(ssems[y], y)
        nxt = jnp.minimum(cur + 1, n_chunks - 1)
        cs, cd, cw = load_idx(nxt, y)
        cs.wait()
        fire_gathers(y)
        cd.wait()
        cw.wait()
        drain(gsems[k], k)
        _scale_rows(rows, wv, k)
        fire_scatters(k)

    _fill_rows(rows, CHUNK, 0.0)
    ob = tid * OUT_PT
    for (o, n) in _out_blocks():
        pltpu.sync_copy(rows.at[0, pl.ds(0, n)], acc.at[pl.ds(ob + o, n)])
    plsc.subcore_barrier()

    cs, cd, cw = load_idx(0, 0)
    cs.wait()
    fire_gathers(0)
    cd.wait()
    cw.wait()
    stage(0, 0, False)
    stage(1, 1, False)
    stage(2, 2, True)

    def triple(g, _):
        stage(3 * g, 0, True)
        stage(3 * g + 1, 1, True)
        stage(3 * g + 2, 2, True)
        return 0

    lax.fori_loop(1, n_chunks // 3, triple, 0)
    drain(gsems[0], 0)
    drain(ssems[(n_chunks - 2) % 3], 0)
    drain(ssems[(n_chunks - 1) % 3], 1)
    plsc.subcore_barrier()

    if relu:
        for (o, n) in _out_blocks():
            pltpu.sync_copy(acc.at[pl.ds(ob + o, n)], rows.at[0, pl.ds(0, n)])
            _relu_rows(rows, n)
            pltpu.sync_copy(rows.at[0, pl.ds(0, n)], out.at[pl.ds(ob + o, n)])
    else:
        pltpu.sync_copy(acc.at[pl.ds(ob, OUT_PT)], out.at[pl.ds(ob, OUT_PT)])


def _sc_scratch():
    return [
        pltpu.VMEM((3, SUB, SEG), jnp.int32),
        pltpu.VMEM((3, SUB, SEG), jnp.int32),
        pltpu.VMEM((3, CHUNK), _f32),
        pltpu.VMEM((3, CHUNK, LN), _f32),
        pltpu.VMEM_SHARED((NP, LN), _f32),
        pltpu.SemaphoreType.DMA,
        pltpu.SemaphoreType.DMA,
        pltpu.SemaphoreType.DMA,
        pltpu.SemaphoreType.DMA,
        pltpu.SemaphoreType.DMA,
        pltpu.SemaphoreType.DMA,
        pltpu.SemaphoreType.DMA,
    ]


_MESH = dict(core_axis_name="c", subcore_axis_name="s")


def _make_spmm12():
    @functools.partial(
        pl.kernel,
        mesh=plsc.VectorSubcoreMesh(**_MESH),
        out_type=[jax.ShapeDtypeStruct((2, NP, LN), _f32),
                  jax.ShapeDtypeStruct((2, NP, LN), _f32)],
        scratch_types=_sc_scratch(),
        compiler_params=pltpu.CompilerParams(use_tc_tiling_on_sc=False),
    )
    def k(src2, dst2, w1d, t, h, s2,
          srcv, dstv, wv, rows, acc, lsem, g0, g1, g2, s0, s1, s2s):
        c = lax.axis_index("c")
        s = lax.axis_index("s")
        base = s * ROWS_L12
        gsems = (g0, g1, g2)
        ssems = (s0, s1, s2s)
        _spmm_core(src2, dst2, w1d, t.at[c], h.at[c], srcv, dstv, wv, rows,
                   acc, lsem, gsems, ssems, s, NCH_L12, base, True)
        _spmm_core(src2, dst2, w1d, h.at[c], s2.at[c], srcv, dstv, wv, rows,
                   acc, lsem, gsems, ssems, s, NCH_L12, base, False)

    return k


def _make_spmm3():
    @functools.partial(
        pl.kernel,
        mesh=plsc.VectorSubcoreMesh(**_MESH),
        out_type=jax.ShapeDtypeStruct((2, NP, LN), _f32),
        scratch_types=_sc_scratch(),
        compiler_params=pltpu.CompilerParams(use_tc_tiling_on_sc=False),
    )
    def k(src2, dst2, w1d, tbl, p,
          srcv, dstv, wv, rows, acc, lsem, g0, g1, g2, s0, s1, s2s):
        c = lax.axis_index("c")
        s = lax.axis_index("s")
        base = (c * 16 + s) * ROWS_L3
        _spmm_core(src2, dst2, w1d, tbl, p.at[c], srcv, dstv, wv, rows,
                   acc, lsem, (g0, g1, g2), (s0, s1, s2s),
                   s, NCH_L3, base, False)

    return k


_spmm12 = _make_spmm12()
_spmm3 = _make_spmm3()

_B1 = 2000
_B2 = 4000


def _mm1_body(x_ref, w_ref, t_ref):
    r = jnp.dot(x_ref[0], w_ref[0], preferred_element_type=_f32)
    r += jnp.dot(x_ref[1], w_ref[1], preferred_element_type=_f32)
    r += jnp.dot(x_ref[2], w_ref[2], preferred_element_type=_f32)
    t_ref[0] = r[:, :LN]
    t_ref[1] = r[:, LN:]


_mm1 = pl.pallas_call(
    _mm1_body,
    grid=(N // _B1,),
    in_specs=[pl.BlockSpec((C, _B1, F), lambda i: (0, i, 0)),
              pl.BlockSpec((C, F, 2 * LN), lambda i: (0, 0, 0))],
    out_specs=pl.BlockSpec((2, _B1, LN), lambda i: (0, i, 0)),
    out_shape=jax.ShapeDtypeStruct((2, NP, LN), _f32),
)


NP8 = NP // 8
_BP = NP8 // 4


def _mm2_body(s2_ref, w2a_ref, w2b_ref, w3_ref, g_ref):
    t = jnp.maximum(
        jnp.dot(s2_ref[0], w2a_ref[...], preferred_element_type=_f32,
                precision=lax.Precision.HIGHEST)
        + jnp.dot(s2_ref[1], w2b_ref[...], preferred_element_type=_f32,
                  precision=lax.Precision.HIGHEST), 0.0)
    g_ref[...] = jnp.dot(t, w3_ref[...], preferred_element_type=_f32,
                         precision=lax.Precision.HIGHEST)


_mm2 = pl.pallas_call(
    _mm2_body,
    grid=(4,),
    in_specs=[pl.BlockSpec((2, _BP, 128), lambda i: (0, i, 0)),
              pl.BlockSpec((128, 8 * H2), lambda i: (0, 0)),
              pl.BlockSpec((128, 8 * H2), lambda i: (0, 0)),
              pl.BlockSpec((8 * H2, 128), lambda i: (0, 0))],
    out_specs=pl.BlockSpec((_BP, 128), lambda i: (i, 0)),
    out_shape=jax.ShapeDtypeStruct((NP8, 128), _f32),
)


def _fin_body(p_ref, o_ref):
    o_ref[...] = p_ref[0] + p_ref[1]


_fin = pl.pallas_call(
    _fin_body,
    grid=(4,),
    in_specs=[pl.BlockSpec((2, _BP, 128), lambda i: (0, i, 0))],
    out_specs=pl.BlockSpec((_BP, 128), lambda i: (i, 0)),
    out_shape=jax.ShapeDtypeStruct((NP8, 128), _f32),
)


def kernel(x, edge_index, edge_weight, W1, W2, W3):
    x2 = x.astype(jnp.bfloat16).transpose(2, 0, 1)
    w1p = jnp.pad(W1.transpose(2, 0, 1),
                  ((0, 0), (0, 0), (0, 2 * LN - H1))).astype(jnp.bfloat16)
    pad = EP - E0
    src2 = jnp.concatenate(
        [edge_index[0], jnp.zeros((pad,), jnp.int32)]).reshape(EP // SEG, SEG)
    dst2 = jnp.concatenate(
        [edge_index[1], jnp.zeros((pad,), jnp.int32)]).reshape(EP // SEG, SEG)
    w1d = jnp.concatenate([edge_weight, jnp.zeros((pad,), _f32)])

    t = _mm1(x2, w1p)
    h, s2 = _spmm12(src2, dst2, w1d, t)
    w2p = jnp.pad(W2, ((0, 2 * LN - H1), (0, 0)))
    w3p = jnp.pad(W3, ((0, 0), (0, LN - OUT)))
    eye8 = jnp.eye(8, dtype=_f32)
    w2a = jnp.kron(eye8, w2p[:LN])
    w2b = jnp.kron(eye8, w2p[LN:])
    w3b = jnp.kron(eye8, w3p)
    g3 = _mm2(s2.reshape(2, NP8, 128), w2a, w2b, w3b).reshape(NP, LN)
    p = _spmm3(src2, dst2, w1d, g3)
    return _fin(p.reshape(2, NP8, 128)).reshape(NP, LN)[:N, :OUT]


# speedup vs baseline: 49.9353x; 1.0684x over previous
import functools

import jax
import jax.numpy as jnp
from jax import lax
from jax.experimental import pallas as pl
from jax.experimental.pallas import tpu as pltpu
from jax.experimental.pallas import tpu_sc as plsc

N = 100000
F, C = 128, 3
H1, H2, OUT = 20, 40, 2
E0 = 3200000
LN = 16
SEG = 128
CHUNK = 512
SUB = CHUNK // SEG
EP = 32 * CHUNK * 198
ROWS_L12 = EP // 16 // SEG
ROWS_L3 = EP // 32 // SEG
NCH_L12 = ROWS_L12 // SUB
NCH_L3 = ROWS_L3 // SUB
NP = 100096
OUT_PT = NP // 16

_f32 = jnp.float32


def _out_blocks():
    blks, o = [], 0
    while o < OUT_PT:
        n = min(CHUNK, OUT_PT - o)
        blks.append((o, n))
        o += n
    return blks


def _fill_rows(rows, n, val):
    def body(i, _):
        rows[0, i, :] = jnp.full((LN,), val, _f32)
        return 0
    lax.fori_loop(0, n, body, 0)


def _scale_rows(rows, wv, b):
    @plsc.parallel_loop(0, CHUNK // LN, unroll=2)
    def body(i):
        w16 = wv[b, pl.ds(i * LN, LN)]
        e0 = i * LN
        for k in range(LN):
            rows[b, e0 + k, :] = rows[b, e0 + k, :] * w16[k]


def _relu_rows(rows, n):
    def body(i, _):
        rows[0, i, :] = jnp.maximum(rows[0, i, :], 0.0)
        return 0
    lax.fori_loop(0, n, body, 0, unroll=8)


def _spmm_core(src2, dst2, w1d, table, out, srcv, dstv, wv, rows, acc,
               lsem, gsems, ssems, tid, n_chunks, edge_row_base, relu):
    def drain(sem, b):
        pltpu.make_async_copy(table.at[pl.ds(0, CHUNK)], rows.at[b], sem).wait()

    def load_idx(chunk_idx, b):
        row0 = edge_row_base + chunk_idx * SUB
        cs = pltpu.async_copy(src2.at[pl.ds(row0, SUB)], srcv.at[b], lsem)
        cd = pltpu.async_copy(dst2.at[pl.ds(row0, SUB)], dstv.at[b], lsem)
        cw = pltpu.async_copy(w1d.at[pl.ds(row0 * SEG, CHUNK)], wv.at[b], lsem)
        return cs, cd, cw

    def fire_gathers(b):
        for j in range(SUB):
            pltpu.async_copy(table.at[srcv.at[b, j]],
                             rows.at[b, pl.ds(j * SEG, SEG)], gsems[b])

    def fire_scatters(b):
        for j in range(SUB):
            pltpu.async_copy(rows.at[b, pl.ds(j * SEG, SEG)],
                             acc.at[dstv.at[b, j]], ssems[b], add=True)

    def stage(cur, k, drain_scatter):
        y = (k + 1) % 3
        if drain_scatter:
            drain(ssems[y], y)
        nxt = jnp.minimum(cur + 1, n_chunks - 1)
        cs, cd, cw = load_idx(nxt, y)
        cs.wait()
        fire_gathers(y)
        cd.wait()
        cw.wait()
        drain(gsems[k], k)
        _scale_rows(rows, wv, k)
        fire_scatters(k)

    _fill_rows(rows, CHUNK, 0.0)
    ob = tid * OUT_PT
    for (o, n) in _out_blocks():
        pltpu.sync_copy(rows.at[0, pl.ds(0, n)], acc.at[pl.ds(ob + o, n)])
    plsc.subcore_barrier()

    cs, cd, cw = load_idx(0, 0)
    cs.wait()
    fire_gathers(0)
    cd.wait()
    cw.wait()
    stage(0, 0, False)
    stage(1, 1, False)
    stage(2, 2, True)

    def triple(g, _):
        stage(3 * g, 0, True)
        stage(3 * g + 1, 1, True)
        stage(3 * g + 2, 2, True)
        return 0

    lax.fori_loop(1, n_chunks // 3, triple, 0)
    drain(gsems[0], 0)
    drain(ssems[(n_chunks - 2) % 3], 0)
    drain(ssems[(n_chunks - 1) % 3], 1)
    plsc.subcore_barrier()

    if relu:
        for (o, n) in _out_blocks():
            pltpu.sync_copy(acc.at[pl.ds(ob + o, n)], rows.at[0, pl.ds(0, n)])
            _relu_rows(rows, n)
            pltpu.sync_copy(rows.at[0, pl.ds(0, n)], out.at[pl.ds(ob + o, n)])
    else:
        pltpu.sync_copy(acc.at[pl.ds(ob, OUT_PT)], out.at[pl.ds(ob, OUT_PT)])


def _sc_scratch():
    return [
        pltpu.VMEM((3, SUB, SEG), jnp.int32),
        pltpu.VMEM((3, SUB, SEG), jnp.int32),
        pltpu.VMEM((3, CHUNK), _f32),
        pltpu.VMEM((3, CHUNK, LN), _f32),
        pltpu.VMEM_SHARED((NP, LN), _f32),
        pltpu.SemaphoreType.DMA,
        pltpu.SemaphoreType.DMA,
        pltpu.SemaphoreType.DMA,
        pltpu.SemaphoreType.DMA,
        pltpu.SemaphoreType.DMA,
        pltpu.SemaphoreType.DMA,
        pltpu.SemaphoreType.DMA,
    ]


_MESH = dict(core_axis_name="c", subcore_axis_name="s")


def _make_spmm12():
    @functools.partial(
        pl.kernel,
        mesh=plsc.VectorSubcoreMesh(**_MESH),
        out_type=[jax.ShapeDtypeStruct((2, NP, LN), _f32),
                  jax.ShapeDtypeStruct((2, NP, LN), _f32)],
        scratch_types=_sc_scratch(),
        compiler_params=pltpu.CompilerParams(use_tc_tiling_on_sc=False),
    )
    def k(src2, dst2, w1d, t, h, s2,
          srcv, dstv, wv, rows, acc, lsem, g0, g1, g2, s0, s1, s2s):
        c = lax.axis_index("c")
        s = lax.axis_index("s")
        base = s * ROWS_L12
        gsems = (g0, g1, g2)
        ssems = (s0, s1, s2s)
        _spmm_core(src2, dst2, w1d, t.at[c], h.at[c], srcv, dstv, wv, rows,
                   acc, lsem, gsems, ssems, s, NCH_L12, base, True)
        _spmm_core(src2, dst2, w1d, h.at[c], s2.at[c], srcv, dstv, wv, rows,
                   acc, lsem, gsems, ssems, s, NCH_L12, base, False)

    return k


def _make_spmm3():
    @functools.partial(
        pl.kernel,
        mesh=plsc.VectorSubcoreMesh(**_MESH),
        out_type=jax.ShapeDtypeStruct((2, NP, LN), _f32),
        scratch_types=_sc_scratch(),
        compiler_params=pltpu.CompilerParams(use_tc_tiling_on_sc=False),
    )
    def k(src2, dst2, w1d, tbl, p,
          srcv, dstv, wv, rows, acc, lsem, g0, g1, g2, s0, s1, s2s):
        c = lax.axis_index("c")
        s = lax.axis_index("s")
        base = (c * 16 + s) * ROWS_L3
        _spmm_core(src2, dst2, w1d, tbl, p.at[c], srcv, dstv, wv, rows,
                   acc, lsem, (g0, g1, g2), (s0, s1, s2s),
                   s, NCH_L3, base, False)

    return k


_spmm12 = _make_spmm12()
_spmm3 = _make_spmm3()

_B1 = 2000
_B2 = 4000


def _mm1_body(x_ref, w_ref, t_ref):
    r = jnp.dot(x_ref[0], w_ref[0], preferred_element_type=_f32)
    r += jnp.dot(x_ref[1], w_ref[1], preferred_element_type=_f32)
    r += jnp.dot(x_ref[2], w_ref[2], preferred_element_type=_f32)
    t_ref[0] = r[:, :LN]
    t_ref[1] = r[:, LN:]


_mm1 = pl.pallas_call(
    _mm1_body,
    grid=(N // _B1,),
    in_specs=[pl.BlockSpec((C, _B1, F), lambda i: (0, i, 0)),
              pl.BlockSpec((C, F, 2 * LN), lambda i: (0, 0, 0))],
    out_specs=pl.BlockSpec((2, _B1, LN), lambda i: (0, i, 0)),
    out_shape=jax.ShapeDtypeStruct((2, NP, LN), _f32),
)


NP8 = NP // 8
_BP = NP8 // 4


def _mm2_body(s2_ref, w2a_ref, w2b_ref, w3_ref, g_ref):
    t = jnp.maximum(
        jnp.dot(s2_ref[0], w2a_ref[...], preferred_element_type=_f32,
                precision=lax.Precision.HIGHEST)
        + jnp.dot(s2_ref[1], w2b_ref[...], preferred_element_type=_f32,
                  precision=lax.Precision.HIGHEST), 0.0)
    g_ref[...] = jnp.dot(t, w3_ref[...], preferred_element_type=_f32,
                         precision=lax.Precision.HIGHEST)


_mm2 = pl.pallas_call(
    _mm2_body,
    grid=(4,),
    in_specs=[pl.BlockSpec((2, _BP, 128), lambda i: (0, i, 0)),
              pl.BlockSpec((128, 8 * H2), lambda i: (0, 0)),
              pl.BlockSpec((128, 8 * H2), lambda i: (0, 0)),
              pl.BlockSpec((8 * H2, 128), lambda i: (0, 0))],
    out_specs=pl.BlockSpec((_BP, 128), lambda i: (i, 0)),
    out_shape=jax.ShapeDtypeStruct((NP8, 128), _f32),
)


def _fin_body(p_ref, o_ref):
    o_ref[...] = p_ref[0] + p_ref[1]


_fin = pl.pallas_call(
    _fin_body,
    grid=(4,),
    in_specs=[pl.BlockSpec((2, _BP, 128), lambda i: (0, i, 0))],
    out_specs=pl.BlockSpec((_BP, 128), lambda i: (i, 0)),
    out_shape=jax.ShapeDtypeStruct((NP8, 128), _f32),
)


def kernel(x, edge_index, edge_weight, W1, W2, W3):
    x2 = x.transpose(2, 0, 1)
    w1p = jnp.pad(W1.transpose(2, 0, 1),
                  ((0, 0), (0, 0), (0, 2 * LN - H1)))
    pad = EP - E0
    src2 = jnp.concatenate(
        [edge_index[0], jnp.zeros((pad,), jnp.int32)]).reshape(EP // SEG, SEG)
    dst2 = jnp.concatenate(
        [edge_index[1], jnp.zeros((pad,), jnp.int32)]).reshape(EP // SEG, SEG)
    w1d = jnp.concatenate([edge_weight, jnp.zeros((pad,), _f32)])

    t = _mm1(x2, w1p)
    h, s2 = _spmm12(src2, dst2, w1d, t)
    w2p = jnp.pad(W2, ((0, 2 * LN - H1), (0, 0)))
    w3p = jnp.pad(W3, ((0, 0), (0, LN - OUT)))
    eye8 = jnp.eye(8, dtype=_f32)
    w2a = jnp.kron(eye8, w2p[:LN])
    w2b = jnp.kron(eye8, w2p[LN:])
    w3b = jnp.kron(eye8, w3p)
    g3 = _mm2(s2.reshape(2, NP8, 128), w2a, w2b, w3b).reshape(NP, LN)
    p = _spmm3(src2, dst2, w1d, g3)
    return _fin(p.reshape(2, NP8, 128)).reshape(NP, LN)[:N, :OUT]
